# baseline (speedup 1.0000x reference)
import functools

import jax
import jax.numpy as jnp
from jax import lax
from jax.experimental import pallas as pl
from jax.experimental.pallas import tpu as pltpu
from jax.experimental.pallas import tpu_sc as plsc

_B = 16384
_D = 64
_NC = 2
_NS = 16
_NW = _NC * _NS
_CHUNK = 128
_ROWS_PER_W = _B // _NW
_NCHUNK = _ROWS_PER_W // _CHUNK
_GROUPS = _CHUNK // 16
_MARGIN = 1.0


def _rsqrt(x):
    x = jnp.maximum(x, 1e-30)
    i = plsc.bitcast(x, jnp.int32)
    i = 0x5F3759DF - lax.shift_right_logical(i, 1)
    y = plsc.bitcast(i, jnp.float32)
    for _ in range(3):
        y = y * (1.5 - 0.5 * x * y * y)
    return y


def _sqrt(x):
    x = jnp.maximum(x, 0.0)
    return x * _rsqrt(x)


def _sc_body(ph_h, pt_h, nh_h, nt_h, rl_h, ent_h, rel_h, out_h,
             ph_i, pt_i, nh_i, nt_i, rl_i,
             hp_r, tp_r, hn_r, tn_r, rr_r, part_v, sem):
    wid = lax.axis_index("s") * _NC + lax.axis_index("c")
    iota = lax.iota(jnp.int32, 16)

    def chunk_body(c, acc):
        base = wid * _ROWS_PER_W + c * _CHUNK
        sl = pl.ds(base, _CHUNK)
        idx_cp = [pltpu.async_copy(ph_h.at[sl], ph_i, sem),
                  pltpu.async_copy(pt_h.at[sl], pt_i, sem),
                  pltpu.async_copy(nh_h.at[sl], nh_i, sem),
                  pltpu.async_copy(nt_h.at[sl], nt_i, sem),
                  pltpu.async_copy(rl_h.at[sl], rl_i, sem)]
        for cp in idx_cp:
            cp.wait()
        row_cp = [pltpu.async_copy(ent_h.at[ph_i], hp_r, sem),
                  pltpu.async_copy(ent_h.at[pt_i], tp_r, sem),
                  pltpu.async_copy(ent_h.at[nh_i], hn_r, sem),
                  pltpu.async_copy(ent_h.at[nt_i], tn_r, sem),
                  pltpu.async_copy(rel_h.at[rl_i], rr_r, sem)]
        for cp in row_cp:
            cp.wait()

        def group_body(g, acc_in):
            rid = g * 16 + iota
            z = jnp.zeros((16,), jnp.float32)
            s_hphp = s_tptp = s_rr = s_hpr = s_hptp = s_rtp = z
            s_hnhn = s_tntn = s_hnr = s_hntn = s_rtn = z
            for j in range(_D):
                cid = lax.bitwise_and(iota + j, _D - 1)
                hp = plsc.load_gather(hp_r, [rid, cid])
                tp = plsc.load_gather(tp_r, [rid, cid])
                hn = plsc.load_gather(hn_r, [rid, cid])
                tn = plsc.load_gather(tn_r, [rid, cid])
                rr = plsc.load_gather(rr_r, [rid, cid])
                s_hphp += hp * hp
                s_tptp += tp * tp
                s_rr += rr * rr
                s_hpr += hp * rr
                s_hptp += hp * tp
                s_rtp += rr * tp
                s_hnhn += hn * hn
                s_tntn += tn * tn
                s_hnr += hn * rr
                s_hntn += hn * tn
                s_rtn += rr * tn
            ar = _rsqrt(s_rr)
            ahp = _rsqrt(s_hphp)
            atp = _rsqrt(s_tptp)
            ahn = _rsqrt(s_hnhn)
            atn = _rsqrt(s_tntn)
            d2p = (s_hphp * ahp * ahp + s_rr * ar * ar + s_tptp * atp * atp
                   + 2.0 * (s_hpr * ahp * ar - s_hptp * ahp * atp
                            - s_rtp * ar * atp))
            d2n = (s_hnhn * ahn * ahn + s_rr * ar * ar + s_tntn * atn * atn
                   + 2.0 * (s_hnr * ahn * ar - s_hntn * ahn * atn
                            - s_rtn * ar * atn))
            e_pos = _sqrt(d2p)
            e_neg = _sqrt(d2n)
            return acc_in + jnp.maximum(_MARGIN + e_pos - e_neg, 0.0)

        return lax.fori_loop(0, _GROUPS, group_body, acc)

    acc = lax.fori_loop(0, _NCHUNK, chunk_body, jnp.zeros((16,), jnp.float32))
    part_v[...] = acc
    pltpu.sync_copy(part_v, out_h.at[wid])


_sc_kernel = pl.kernel(
    _sc_body,
    out_type=jax.ShapeDtypeStruct((_NW, 16), jnp.float32),
    mesh=plsc.VectorSubcoreMesh(core_axis_name="c", subcore_axis_name="s",
                                num_cores=_NC, num_subcores=_NS),
    scratch_types=[pltpu.VMEM((_CHUNK,), jnp.int32) for _ in range(5)]
    + [pltpu.VMEM((_CHUNK, _D), jnp.float32) for _ in range(5)]
    + [pltpu.VMEM((16,), jnp.float32), pltpu.SemaphoreType.DMA],
    compiler_params=pltpu.CompilerParams(needs_layout_passes=False,
                                         use_tc_tiling_on_sc=False),
)


def _final_body(p_ref, o_ref):
    o_ref[...] = jnp.reshape(jnp.sum(p_ref[...]) * (1.0 / _B), (1, 1))


def kernel(pos_pairs, neg_pairs, rels, ent_emb, rel_emb):
    ph = pos_pairs[:, 0].astype(jnp.int32)
    pt = pos_pairs[:, 1].astype(jnp.int32)
    nh = neg_pairs[:, 0].astype(jnp.int32)
    nt = neg_pairs[:, 1].astype(jnp.int32)
    rl = rels[:, 0].astype(jnp.int32)
    partials = _sc_kernel(ph, pt, nh, nt, rl, ent_emb, rel_emb)
    total = pl.pallas_call(
        _final_body,
        out_shape=jax.ShapeDtypeStruct((1, 1), jnp.float32),
    )(partials)
    return total[0, 0]


# speedup vs baseline: 1.2705x; 1.2705x over previous
import jax
import jax.numpy as jnp
from jax import lax
from jax.experimental import pallas as pl
from jax.experimental.pallas import tpu as pltpu
from jax.experimental.pallas import tpu_sc as plsc

_B = 16384
_D = 64
_NC = 2
_NS = 16
_NW = _NC * _NS
_ROWS_PER_W = _B // _NW
_CHUNK = 16
_NCHUNK = _ROWS_PER_W // _CHUNK
_MARGIN = 1.0


def _rsqrt(x):
    x = jnp.maximum(x, 1e-30)
    i = plsc.bitcast(x, jnp.int32)
    i = 0x5F3759DF - lax.shift_right_logical(i, 1)
    y = plsc.bitcast(i, jnp.float32)
    for _ in range(3):
        y = y * (1.5 - 0.5 * x * y * y)
    return y


def _sqrt(x):
    x = jnp.maximum(x, 0.0)
    return x * _rsqrt(x)


def _sc_body(ph_h, pt_h, nh_h, nt_h, rl_h, ent_h, rel_h, out_h,
             ph_i, pt_i, nh_i, nt_i, rl_i,
             hp_r, tp_r, hn_r, tn_r, rr_r, part_v, sem):
    wid = lax.axis_index("s") * _NC + lax.axis_index("c")
    iota = lax.iota(jnp.int32, 16)
    base = wid * _ROWS_PER_W
    sl = pl.ds(base, _ROWS_PER_W)
    idx_cp = [pltpu.async_copy(ph_h.at[sl], ph_i, sem),
              pltpu.async_copy(pt_h.at[sl], pt_i, sem),
              pltpu.async_copy(nh_h.at[sl], nh_i, sem),
              pltpu.async_copy(nt_h.at[sl], nt_i, sem),
              pltpu.async_copy(rl_h.at[sl], rl_i, sem)]
    for cp in idx_cp:
        cp.wait()

    def chunk_body(c, acc):
        csl = pl.ds(c * _CHUNK, _CHUNK)
        ivs = [ph_i[csl], pt_i[csl], nh_i[csl], nt_i[csl], rl_i[csl]]
        bufs = [hp_r, tp_r, hn_r, tn_r, rr_r]
        tabs = [ent_h, ent_h, ent_h, ent_h, rel_h]
        cps = []
        for iv, buf, tab in zip(ivs, bufs, tabs):
            blk = lax.shift_right_logical(iv, 3)
            for k in range(_CHUNK):
                cps.append(pltpu.async_copy(
                    tab.at[pl.ds(blk[k] * 8, 8)], buf.at[k], sem))
        for cp in cps:
            cp.wait()

        offs = [lax.bitwise_and(iv, 7) for iv in ivs]
        z = jnp.zeros((16,), jnp.float32)
        s_hphp = s_tptp = s_rr = s_hpr = s_hptp = s_rtp = z
        s_hnhn = s_tntn = s_hnr = s_hntn = s_rtn = z
        for j in range(_D):
            cid = lax.bitwise_and(iota + j, _D - 1)
            hp = plsc.load_gather(hp_r, [iota, offs[0], cid])
            tp = plsc.load_gather(tp_r, [iota, offs[1], cid])
            hn = plsc.load_gather(hn_r, [iota, offs[2], cid])
            tn = plsc.load_gather(tn_r, [iota, offs[3], cid])
            rr = plsc.load_gather(rr_r, [iota, offs[4], cid])
            s_hphp += hp * hp
            s_tptp += tp * tp
            s_rr += rr * rr
            s_hpr += hp * rr
            s_hptp += hp * tp
            s_rtp += rr * tp
            s_hnhn += hn * hn
            s_tntn += tn * tn
            s_hnr += hn * rr
            s_hntn += hn * tn
            s_rtn += rr * tn
        ar = _rsqrt(s_rr)
        ahp = _rsqrt(s_hphp)
        atp = _rsqrt(s_tptp)
        ahn = _rsqrt(s_hnhn)
        atn = _rsqrt(s_tntn)
        d2p = (s_hphp * ahp * ahp + s_rr * ar * ar + s_tptp * atp * atp
               + 2.0 * (s_hpr * ahp * ar - s_hptp * ahp * atp
                        - s_rtp * ar * atp))
        d2n = (s_hnhn * ahn * ahn + s_rr * ar * ar + s_tntn * atn * atn
               + 2.0 * (s_hnr * ahn * ar - s_hntn * ahn * atn
                        - s_rtn * ar * atn))
        e_pos = _sqrt(d2p)
        e_neg = _sqrt(d2n)
        return acc + jnp.maximum(_MARGIN + e_pos - e_neg, 0.0)

    acc = lax.fori_loop(0, _NCHUNK, chunk_body, jnp.zeros((16,), jnp.float32))
    part_v[...] = acc
    pltpu.sync_copy(part_v, out_h.at[wid])


_sc_kernel = pl.kernel(
    _sc_body,
    out_type=jax.ShapeDtypeStruct((_NW, 16), jnp.float32),
    mesh=plsc.VectorSubcoreMesh(core_axis_name="c", subcore_axis_name="s",
                                num_cores=_NC, num_subcores=_NS),
    scratch_types=[pltpu.VMEM((_ROWS_PER_W,), jnp.int32) for _ in range(5)]
    + [pltpu.VMEM((_CHUNK, 8, _D), jnp.float32) for _ in range(5)]
    + [pltpu.VMEM((16,), jnp.float32), pltpu.SemaphoreType.DMA],
    compiler_params=pltpu.CompilerParams(needs_layout_passes=False,
                                         use_tc_tiling_on_sc=True),
)


def _final_body(p_ref, o_ref):
    o_ref[...] = jnp.reshape(jnp.sum(p_ref[...]) * (1.0 / _B), (1, 1))


def kernel(pos_pairs, neg_pairs, rels, ent_emb, rel_emb):
    ph = pos_pairs[:, 0].astype(jnp.int32)
    pt = pos_pairs[:, 1].astype(jnp.int32)
    nh = neg_pairs[:, 0].astype(jnp.int32)
    nt = neg_pairs[:, 1].astype(jnp.int32)
    rl = rels[:, 0].astype(jnp.int32)
    partials = _sc_kernel(ph, pt, nh, nt, rl, ent_emb, rel_emb)
    total = pl.pallas_call(
        _final_body,
        out_shape=jax.ShapeDtypeStruct((1, 1), jnp.float32),
    )(partials)
    return total[0, 0]
